# baseline (speedup 1.0000x reference)
import functools

import jax
import jax.numpy as jnp
from jax import lax
from jax.experimental import pallas as pl
from jax.experimental.pallas import tpu as pltpu
from jax.experimental.pallas import tpu_sc as plsc

_B, _Z, _M, _T = 1024, 32, 8, 256
_NC, _NS, _L = 2, 16, 16
_NW = _NC * _NS
_OPW = _B * _Z * _M // _NW
_BPW = _B // _NW
_QPW = _BPW * _M
_NQ = 8
_BPQ = _BPW // _NQ
_EPQ = _OPW // _NQ
_RPQ = _EPQ // _M


def _tec_body(tin_hbm, tbg_hbm, src_hbm, out_hbm,
              tbg_v, tin_v, tidx_v, idx_v, gat_v, pad_a, pad_b,
              gsem0, gsem1, gsem2, gsem3, gsem4, gsem5, gsem6, gsem7,
              osem_a, osem_b):
    wid = lax.axis_index("s") * _NC + lax.axis_index("c")
    obase = wid * _OPW

    pltpu.async_copy(tbg_hbm, tbg_v, osem_a)
    pltpu.async_copy(tin_hbm.at[pl.ds(wid * _QPW, _QPW)], tin_v, osem_b)
    pltpu.make_async_copy(tbg_hbm, tbg_v, osem_a).wait()
    pltpu.make_async_copy(tin_hbm.at[pl.ds(wid * _QPW, _QPW)], tin_v,
                          osem_b).wait()

    lane = lax.iota(jnp.int32, _L)

    def q_step(i, _):
        t = tin_v[pl.ds(i * _L, _L)]
        lo = jnp.zeros((_L,), jnp.int32)
        hi = jnp.full((_L,), _T, jnp.int32)

        def bs(_k, carry):
            lo, hi = carry
            mid = (lo + hi) >> 1
            a = plsc.load_gather(tbg_v, [jnp.minimum(mid, _T - 1)])
            below = a < t
            return (jnp.where(below, mid + 1, lo),
                    jnp.where(below, hi, mid))

        lo, hi = lax.fori_loop(0, 9, bs, (lo, hi))
        tidx_v[pl.ds(i * _L, _L)] = jnp.minimum(lo, _T - 1)
        return 0

    lax.fori_loop(0, _QPW // _L, q_step, 0, unroll=2)

    pat_m = lane & (_M - 1)
    pat_r = (lane >> 3) * 2048 + pat_m * 128

    gsems = (gsem0, gsem1, gsem2, gsem3, gsem4, gsem5, gsem6, gsem7)

    for q in range(_NQ):
        def b_step(i, _, q=q):
            bl = q * _BPQ + i
            tq = plsc.load_gather(tidx_v, [pat_m + bl * _M])
            patb = pat_r + ((tq >> 7) << 10) + (tq & 127)
            row0 = (obase + bl * (_Z * _M)) * _T

            def s_step(k, _):
                idx_v[pl.ds(bl * (_Z * _M) + k * _L, _L)] = (
                    patb + (row0 + k * _L * _T))
                return 0

            lax.fori_loop(0, (_Z * _M) // _L, s_step, 0, unroll=4)
            return 0

        lax.fori_loop(0, _BPQ, b_step, 0)
        pltpu.async_copy(src_hbm.at[idx_v.at[pl.ds(q * _EPQ, _EPQ)]],
                         gat_v.at[pl.ds(q * _EPQ, _EPQ)], gsems[q])

    r0 = wid * (_OPW // _M)
    pads = (pad_a, pad_b)
    osems = (osem_a, osem_b)

    for q in range(_NQ):
        pltpu.make_async_copy(src_hbm.at[idx_v.at[pl.ds(q * _EPQ, _EPQ)]],
                              gat_v.at[pl.ds(q * _EPQ, _EPQ)],
                              gsems[q]).wait()
        pb, osem = pads[q % 2], osems[q % 2]
        if q >= 2:
            pltpu.make_async_copy(
                pb, out_hbm.at[pl.ds(r0 + (q - 2) * _RPQ, _RPQ), :],
                osem).wait()

        def row_step(rl, _, q=q, pb=pb):
            pb[rl, pl.ds(0, _L)] = gat_v[pl.ds(q * _EPQ + rl * _M, _L)]
            return 0

        lax.fori_loop(0, _RPQ, row_step, 0, unroll=4)
        pltpu.async_copy(pb, out_hbm.at[pl.ds(r0 + q * _RPQ, _RPQ), :], osem)

    for q in (_NQ - 2, _NQ - 1):
        pltpu.make_async_copy(pads[q % 2],
                              out_hbm.at[pl.ds(r0 + q * _RPQ, _RPQ), :],
                              osems[q % 2]).wait()


_smoother_sc = functools.partial(
    pl.kernel,
    out_type=jax.ShapeDtypeStruct((_B * _Z, 128), jnp.float32),
    mesh=plsc.VectorSubcoreMesh(core_axis_name="c", subcore_axis_name="s"),
    compiler_params=pltpu.CompilerParams(needs_layout_passes=False),
    scratch_types=[
        pltpu.VMEM((_T,), jnp.float32),
        pltpu.VMEM((_QPW,), jnp.float32),
        pltpu.VMEM((_QPW,), jnp.int32),
        pltpu.VMEM((_OPW,), jnp.int32),
        pltpu.VMEM((_OPW + _L,), jnp.float32),
        pltpu.VMEM((_RPQ, 128), jnp.float32),
        pltpu.VMEM((_RPQ, 128), jnp.float32),
        pltpu.SemaphoreType.DMA,
        pltpu.SemaphoreType.DMA,
        pltpu.SemaphoreType.DMA,
        pltpu.SemaphoreType.DMA,
        pltpu.SemaphoreType.DMA,
        pltpu.SemaphoreType.DMA,
        pltpu.SemaphoreType.DMA,
        pltpu.SemaphoreType.DMA,
        pltpu.SemaphoreType.DMA,
        pltpu.SemaphoreType.DMA,
    ],
)(_tec_body)


def kernel(surv_steps, time_bg, time_in, z_smp_n):
    del z_smp_n
    src = (surv_steps.reshape(_B, _Z, _M, _T // 128, 128)
           .transpose(0, 1, 3, 2, 4).reshape(-1))
    out = _smoother_sc(time_in.reshape(-1), time_bg, src)
    out = out.reshape(_B, _Z // 8, 8, 128)
    return out[:, :, :, :_M].reshape(_B, _Z, _M)


# speedup vs baseline: 1.0055x; 1.0055x over previous
import functools

import jax
import jax.numpy as jnp
from jax import lax
from jax.experimental import pallas as pl
from jax.experimental.pallas import tpu as pltpu
from jax.experimental.pallas import tpu_sc as plsc

_B, _Z, _M, _T = 1024, 32, 8, 256
_NC, _NS, _L = 2, 16, 16
_NW = _NC * _NS
_OPW = _B * _Z * _M // _NW
_BPW = _B // _NW
_QPW = _BPW * _M
_NQ = 4
_BPQ = _BPW // _NQ
_EPQ = _OPW // _NQ
_RPQ = _EPQ // _M


def _tec_body(tin_hbm, tbg_hbm, src_hbm, out_hbm,
              tbg_v, tin_v, tidx_v, idx_v, gat_v, pad_a, pad_b,
              gsem0, gsem1, gsem2, gsem3, osem_a, osem_b):
    wid = lax.axis_index("s") * _NC + lax.axis_index("c")
    obase = wid * _OPW

    pltpu.async_copy(tbg_hbm, tbg_v, osem_a)
    pltpu.async_copy(tin_hbm.at[pl.ds(wid * _QPW, _QPW)], tin_v, osem_b)
    pltpu.make_async_copy(tbg_hbm, tbg_v, osem_a).wait()
    pltpu.make_async_copy(tin_hbm.at[pl.ds(wid * _QPW, _QPW)], tin_v,
                          osem_b).wait()

    lane = lax.iota(jnp.int32, _L)

    def q_step(i, _):
        t = tin_v[pl.ds(i * _L, _L)]
        lo = jnp.zeros((_L,), jnp.int32)
        hi = jnp.full((_L,), _T, jnp.int32)

        def bs(_k, carry):
            lo, hi = carry
            mid = (lo + hi) >> 1
            a = plsc.load_gather(tbg_v, [jnp.minimum(mid, _T - 1)])
            below = a < t
            return (jnp.where(below, mid + 1, lo),
                    jnp.where(below, hi, mid))

        lo, hi = lax.fori_loop(0, 9, bs, (lo, hi))
        tidx_v[pl.ds(i * _L, _L)] = jnp.minimum(lo, _T - 1)
        return 0

    lax.fori_loop(0, _QPW // _L, q_step, 0, unroll=2)

    pat_m = lane & (_M - 1)
    pat_r = (lane >> 3) * 2048 + pat_m * 128

    gsems = (gsem0, gsem1, gsem2, gsem3)

    for q in range(_NQ):
        def b_step(i, _, q=q):
            bl = q * _BPQ + i
            tq = plsc.load_gather(tidx_v, [pat_m + bl * _M])
            patb = pat_r + ((tq >> 7) << 10) + (tq & 127)
            row0 = (obase + bl * (_Z * _M)) * _T

            def s_step(k, _):
                idx_v[pl.ds(bl * (_Z * _M) + k * _L, _L)] = (
                    patb + (row0 + k * _L * _T))
                return 0

            lax.fori_loop(0, (_Z * _M) // _L, s_step, 0, unroll=4)
            return 0

        lax.fori_loop(0, _BPQ, b_step, 0)
        pltpu.async_copy(src_hbm.at[idx_v.at[pl.ds(q * _EPQ, _EPQ)]],
                         gat_v.at[pl.ds(q * _EPQ, _EPQ)], gsems[q])

    r0 = wid * (_OPW // _M)
    pads = (pad_a, pad_b)
    osems = (osem_a, osem_b)

    for q in range(_NQ):
        pltpu.make_async_copy(src_hbm.at[idx_v.at[pl.ds(q * _EPQ, _EPQ)]],
                              gat_v.at[pl.ds(q * _EPQ, _EPQ)],
                              gsems[q]).wait()
        pb, osem = pads[q % 2], osems[q % 2]
        if q >= 2:
            pltpu.make_async_copy(
                pb, out_hbm.at[pl.ds(r0 + (q - 2) * _RPQ, _RPQ), :],
                osem).wait()

        def row_step(rl, _, q=q, pb=pb):
            pb[rl, pl.ds(0, _L)] = gat_v[pl.ds(q * _EPQ + rl * _M, _L)]
            return 0

        lax.fori_loop(0, _RPQ, row_step, 0, unroll=4)
        pltpu.async_copy(pb, out_hbm.at[pl.ds(r0 + q * _RPQ, _RPQ), :], osem)

    for q in (_NQ - 2, _NQ - 1):
        pltpu.make_async_copy(pads[q % 2],
                              out_hbm.at[pl.ds(r0 + q * _RPQ, _RPQ), :],
                              osems[q % 2]).wait()


_smoother_sc = functools.partial(
    pl.kernel,
    out_type=jax.ShapeDtypeStruct((_B * _Z, 128), jnp.float32),
    mesh=plsc.VectorSubcoreMesh(core_axis_name="c", subcore_axis_name="s"),
    compiler_params=pltpu.CompilerParams(needs_layout_passes=False),
    scratch_types=[
        pltpu.VMEM((_T,), jnp.float32),
        pltpu.VMEM((_QPW,), jnp.float32),
        pltpu.VMEM((_QPW,), jnp.int32),
        pltpu.VMEM((_OPW,), jnp.int32),
        pltpu.VMEM((_OPW + _L,), jnp.float32),
        pltpu.VMEM((_RPQ, 128), jnp.float32),
        pltpu.VMEM((_RPQ, 128), jnp.float32),
        pltpu.SemaphoreType.DMA,
        pltpu.SemaphoreType.DMA,
        pltpu.SemaphoreType.DMA,
        pltpu.SemaphoreType.DMA,
        pltpu.SemaphoreType.DMA,
        pltpu.SemaphoreType.DMA,
    ],
)(_tec_body)


def kernel(surv_steps, time_bg, time_in, z_smp_n):
    del z_smp_n
    src = (surv_steps.reshape(_B, _Z, _M, _T // 128, 128)
           .transpose(0, 1, 3, 2, 4).reshape(-1))
    out = _smoother_sc(time_in.reshape(-1), time_bg, src)
    out = out.reshape(_B, _Z // 8, 8, 128)
    return out[:, :, :, :_M].reshape(_B, _Z, _M)
